# baseline (speedup 1.0000x reference)
import functools

import jax
import jax.numpy as jnp
from jax import lax
from jax.experimental import pallas as pl
from jax.experimental.pallas import tpu as pltpu
from jax.experimental.pallas import tpu_sc as plsc

K = 8
BQ = 128
CH = 128
NW = 32
SVCAP = 16384 + 32


def _quat_rotmat(q):
    q = q / (jnp.linalg.norm(q, axis=-1, keepdims=True) + 1e-12)
    w, x, y, z = q[..., 0], q[..., 1], q[..., 2], q[..., 3]
    R = jnp.stack([
        1 - 2 * (y * y + z * z), 2 * (x * y - w * z), 2 * (x * z + w * y),
        2 * (x * y + w * z), 1 - 2 * (x * x + z * z), 2 * (y * z - w * x),
        2 * (x * z - w * y), 2 * (y * z + w * x), 1 - 2 * (x * x + y * y)
    ], axis=-1).reshape(q.shape[:-1] + (3, 3))
    return R


def _dist_body(qfeat_ref, q_ref, sflat_ref, b_ref, c_ref, dist_ref, mc_ref):
    n = sflat_ref.shape[0]
    term1 = lax.dot_general(qfeat_ref[...].astype(jnp.bfloat16),
                            sflat_ref[...],
                            (((1,), (1,)), ((), ())),
                            preferred_element_type=jnp.float32)
    term2 = -2.0 * lax.dot_general(q_ref[...].astype(jnp.bfloat16),
                                   b_ref[...].astype(jnp.bfloat16),
                                   (((1,), (1,)), ((), ())),
                                   preferred_element_type=jnp.float32)
    dist = (term1 + term2) + c_ref[...]
    dist_ref[...] = dist
    mc_ref[...] = jnp.min(dist.reshape(BQ, n // CH, CH), axis=2)


def _sc_topk_body(dist_hbm, mc_hbm, vals_hbm, idx_hbm,
                  mc_v, mcwork, cand_rows, cand_chunks, rows_v,
                  sv_v, si_v, vstage, istage, sem, sem2):
    q_total = mc_hbm.shape[0]
    nch = mc_hbm.shape[1]
    qpw = q_total // NW
    nvc = nch // 16
    wid = lax.axis_index("s") * 2 + lax.axis_index("c")
    qbase = wid * qpw
    inf = jnp.float32(jnp.inf)
    iota = lax.iota(jnp.int32, 16)

    pltpu.async_copy(mc_hbm.at[pl.ds(qbase, qpw)], mc_v, sem).wait()
    for v in range((nch + 16) // 16):
        cand_rows[pl.ds(v * 16, 16)] = jnp.zeros((16,), jnp.int32)
        cand_chunks[pl.ds(v * 16, 16)] = jnp.zeros((16,), jnp.int32)

    def per_query(ql, _):
        qid = qbase + ql
        mrow = [mc_v[ql, pl.ds(v * 16, 16)] for v in range(nvc)]
        vm = mrow[0]
        for v in range(1, nvc):
            vm = jnp.minimum(vm, mrow[v])
        t = jnp.sort(vm)[K - 1]

        nc = jnp.int32(0)
        for v in range(nvc):
            mask = mrow[v] <= t
            chunks = v * 16 + iota
            plsc.store_compressed(cand_chunks.at[pl.ds(nc, 16)], chunks, mask=mask)
            plsc.store_compressed(cand_rows.at[pl.ds(nc, 16)],
                                  qid * nch + chunks, mask=mask)
            nc = nc + plsc.all_reduce_population_count(mask)[0]

        nb = (nc + 15) // 16

        def batch_body(bi, scnt):
            ids = cand_rows[pl.ds(bi * 16, 16)]
            pltpu.async_copy(dist_hbm.at[ids], rows_v, sem2).wait()
            nrows = jnp.minimum(jnp.int32(16), nc - bi * 16)

            def row_body(j, scnt):
                chunkid = cand_chunks[pl.ds(bi * 16 + j, 16)][0]
                base = chunkid * CH
                for l in range(CH // 16):
                    dv = rows_v[j, pl.ds(l * 16, 16)]
                    mask = dv <= t
                    plsc.store_compressed(sv_v.at[pl.ds(scnt, 16)], dv,
                                          mask=mask)
                    plsc.store_compressed(si_v.at[pl.ds(scnt, 16)],
                                          base + l * 16 + iota, mask=mask)
                    scnt = scnt + plsc.all_reduce_population_count(mask)[0]
                return scnt

            return lax.fori_loop(0, nrows, row_body, scnt)

        scnt = lax.fori_loop(0, nb, batch_body, jnp.int32(0))

        nv = (scnt + 15) // 16
        sv_v[pl.ds(scnt, 16)] = jnp.full((16,), jnp.inf, jnp.float32)

        def extract(k, st):
            v_out, i_out = st

            def min_body(v, mvec):
                return jnp.minimum(mvec, sv_v[pl.ds(v * 16, 16)])

            mvec = lax.fori_loop(0, nv, min_body,
                                 jnp.full((16,), jnp.inf, jnp.float32))
            m = jnp.min(mvec)

            def f_body(v, pos):
                eq = sv_v[pl.ds(v * 16, 16)] == m
                cnt = plsc.all_reduce_population_count(eq)[0]
                lane = plsc.all_reduce_ffs(eq)[0]
                return jnp.where(jnp.logical_and(pos < 0, cnt > 0),
                                 v * 16 + lane, pos)

            pos = lax.fori_loop(0, nv, f_body, jnp.int32(-1))
            gidx = si_v[pl.ds(pos, 16)][0]
            old = sv_v[pl.ds(pos, 16)]
            sv_v[pl.ds(pos, 16)] = jnp.where(iota == 0, inf, old)
            v_out = jnp.where(iota == k, m, v_out)
            i_out = jnp.where(iota == k, gidx, i_out)
            return v_out, i_out

        v_out, i_out = lax.fori_loop(
            0, K, extract,
            (jnp.full((16,), jnp.inf, jnp.float32),
             jnp.zeros((16,), jnp.int32)))
        vstage[ql, pl.ds(0, 16)] = v_out
        istage[ql, pl.ds(0, 16)] = i_out
        return 0

    lax.fori_loop(0, qpw, per_query, 0)

    pltpu.async_copy(vstage, vals_hbm.at[pl.ds(qbase, qpw)], sem).wait()
    pltpu.async_copy(istage, idx_hbm.at[pl.ds(qbase, qpw)], sem).wait()


def kernel(queries, positions, scales, quaternions):
    Q = queries.shape[0]
    N = positions.shape[0]
    Rm = _quat_rotmat(quaternions)
    s2inv = 1.0 / (scales * scales + 1e-8)
    A = Rm * s2inv[:, None, :]
    Abf = A.astype(jnp.bfloat16)
    Rbf = Rm.astype(jnp.bfloat16)
    Sinv = jnp.einsum('nij,nkj->nik', Rbf, Abf,
                      preferred_element_type=jnp.float32)
    Sflat_bf = Sinv.astype(jnp.bfloat16).reshape(N, 9)
    bt = [Sinv[:, :, j] * positions[:, None, j] for j in range(3)]
    b = (bt[0] + bt[1]) + bt[2]
    ct = [positions[:, j] * b[:, j] for j in range(3)]
    c = (ct[0] + ct[2]) + ct[1]
    Qfeat = (queries[:, :, None] * queries[:, None, :]).reshape(Q, 9)

    nch = N // CH
    HALVES = 4
    Qh = Q // HALVES
    qpw = Qh // NW

    tc = pl.pallas_call(
        _dist_body,
        grid=(Qh // BQ,),
        in_specs=[
            pl.BlockSpec((BQ, 9), lambda i: (i, 0)),
            pl.BlockSpec((BQ, 3), lambda i: (i, 0)),
            pl.BlockSpec((N, 9), lambda i: (0, 0)),
            pl.BlockSpec((N, 3), lambda i: (0, 0)),
            pl.BlockSpec((1, N), lambda i: (0, 0)),
        ],
        out_specs=[
            pl.BlockSpec((BQ, N), lambda i: (i, 0)),
            pl.BlockSpec((BQ, nch), lambda i: (i, 0)),
        ],
        out_shape=[
            jax.ShapeDtypeStruct((Qh, N), jnp.float32),
            jax.ShapeDtypeStruct((Qh, nch), jnp.float32),
        ],
    )

    sc = functools.partial(
        pl.kernel,
        mesh=plsc.VectorSubcoreMesh(core_axis_name="c", subcore_axis_name="s"),
        compiler_params=pltpu.CompilerParams(needs_layout_passes=False),
        out_type=[jax.ShapeDtypeStruct((Qh, 16), jnp.float32),
                  jax.ShapeDtypeStruct((Qh, 16), jnp.int32)],
        scratch_types=[
            pltpu.VMEM((qpw, nch), jnp.float32),
            pltpu.VMEM((nch,), jnp.float32),
            pltpu.VMEM((nch + 16,), jnp.int32),
            pltpu.VMEM((nch + 16,), jnp.int32),
            pltpu.VMEM((16, CH), jnp.float32),
            pltpu.VMEM((SVCAP,), jnp.float32),
            pltpu.VMEM((SVCAP,), jnp.int32),
            pltpu.VMEM((qpw, 16), jnp.float32),
            pltpu.VMEM((qpw, 16), jnp.int32),
            pltpu.SemaphoreType.DMA,
            pltpu.SemaphoreType.DMA,
        ],
    )(_sc_topk_body)

    halves = [tc(Qfeat[h * Qh:(h + 1) * Qh], queries[h * Qh:(h + 1) * Qh],
                 Sflat_bf, b, c[None, :]) for h in range(HALVES)]
    outs = [sc(d.reshape(Qh * nch, CH), m) for d, m in halves]
    vals = jnp.concatenate([o[0][:, :K] for o in outs], axis=0)
    idx = jnp.concatenate([o[1][:, :K] for o in outs], axis=0)
    return vals, idx


# speedup vs baseline: 1.0862x; 1.0862x over previous
import functools

import jax
import jax.numpy as jnp
from jax import lax
from jax.experimental import pallas as pl
from jax.experimental.pallas import tpu as pltpu
from jax.experimental.pallas import tpu_sc as plsc

K = 8
BQ = 128
CH = 128
NW = 32
SVCAP = 16384 + 32


def _quat_rotmat(q):
    q = q / (jnp.linalg.norm(q, axis=-1, keepdims=True) + 1e-12)
    w, x, y, z = q[..., 0], q[..., 1], q[..., 2], q[..., 3]
    R = jnp.stack([
        1 - 2 * (y * y + z * z), 2 * (x * y - w * z), 2 * (x * z + w * y),
        2 * (x * y + w * z), 1 - 2 * (x * x + z * z), 2 * (y * z - w * x),
        2 * (x * z - w * y), 2 * (y * z + w * x), 1 - 2 * (x * x + y * y)
    ], axis=-1).reshape(q.shape[:-1] + (3, 3))
    return R


def _dist_body(qfeat_ref, q_ref, sflat_ref, b_ref, c_ref, dist_ref, mc_ref):
    n = sflat_ref.shape[0]
    term1 = lax.dot_general(qfeat_ref[...].astype(jnp.bfloat16),
                            sflat_ref[...],
                            (((1,), (1,)), ((), ())),
                            preferred_element_type=jnp.float32)
    term2 = -2.0 * lax.dot_general(q_ref[...].astype(jnp.bfloat16),
                                   b_ref[...].astype(jnp.bfloat16),
                                   (((1,), (1,)), ((), ())),
                                   preferred_element_type=jnp.float32)
    dist = (term1 + term2) + c_ref[...]
    dist_ref[...] = dist
    mc_ref[...] = jnp.min(dist.reshape(BQ, n // CH, CH), axis=2)


def _sc_topk_body(dist_hbm, mc_hbm, vals_hbm, idx_hbm,
                  mc_v, mcwork, cand_rows, cand_chunks, rows_v,
                  sv_v, si_v, vstage, istage, sem, sem2):
    q_total = mc_hbm.shape[0]
    nch = mc_hbm.shape[1]
    qpw = q_total // NW
    nvc = nch // 16
    wid = lax.axis_index("s") * 2 + lax.axis_index("c")
    qbase = wid * qpw
    inf = jnp.float32(jnp.inf)
    iota = lax.iota(jnp.int32, 16)

    pltpu.async_copy(mc_hbm.at[pl.ds(qbase, qpw)], mc_v, sem).wait()
    for v in range((nch + 16) // 16):
        cand_rows[pl.ds(v * 16, 16)] = jnp.zeros((16,), jnp.int32)
        cand_chunks[pl.ds(v * 16, 16)] = jnp.zeros((16,), jnp.int32)

    def per_query(ql, _):
        qid = qbase + ql
        mrow = [mc_v[ql, pl.ds(v * 16, 16)] for v in range(nvc)]
        vm = mrow[0]
        for v in range(1, nvc):
            vm = jnp.minimum(vm, mrow[v])
        t = jnp.sort(vm)[K - 1]

        nc = jnp.int32(0)
        for v in range(nvc):
            mask = mrow[v] <= t
            chunks = v * 16 + iota
            plsc.store_compressed(cand_chunks.at[pl.ds(nc, 16)], chunks, mask=mask)
            plsc.store_compressed(cand_rows.at[pl.ds(nc, 16)],
                                  qid * nch + chunks, mask=mask)
            nc = nc + plsc.all_reduce_population_count(mask)[0]

        nb = (nc + 15) // 16

        def batch_body(bi, scnt):
            ids = cand_rows[pl.ds(bi * 16, 16)]
            pltpu.async_copy(dist_hbm.at[ids], rows_v, sem2).wait()
            nrows = jnp.minimum(jnp.int32(16), nc - bi * 16)

            def row_body(j, scnt):
                chunkid = cand_chunks[pl.ds(bi * 16 + j, 16)][0]
                base = chunkid * CH
                for l in range(CH // 16):
                    dv = rows_v[j, pl.ds(l * 16, 16)]
                    mask = dv <= t
                    plsc.store_compressed(sv_v.at[pl.ds(scnt, 16)], dv,
                                          mask=mask)
                    plsc.store_compressed(si_v.at[pl.ds(scnt, 16)],
                                          base + l * 16 + iota, mask=mask)
                    scnt = scnt + plsc.all_reduce_population_count(mask)[0]
                return scnt

            return lax.fori_loop(0, nrows, row_body, scnt)

        scnt = lax.fori_loop(0, nb, batch_body, jnp.int32(0))

        nv = (scnt + 15) // 16
        sv_v[pl.ds(scnt, 16)] = jnp.full((16,), jnp.inf, jnp.float32)

        def extract(k, st):
            v_out, i_out = st

            def min_body(v, mvec):
                return jnp.minimum(mvec, sv_v[pl.ds(v * 16, 16)])

            mvec = lax.fori_loop(0, nv, min_body,
                                 jnp.full((16,), jnp.inf, jnp.float32))
            m = jnp.min(mvec)

            def f_body(v, pos):
                eq = sv_v[pl.ds(v * 16, 16)] == m
                cnt = plsc.all_reduce_population_count(eq)[0]
                lane = plsc.all_reduce_ffs(eq)[0]
                return jnp.where(jnp.logical_and(pos < 0, cnt > 0),
                                 v * 16 + lane, pos)

            pos = lax.fori_loop(0, nv, f_body, jnp.int32(-1))
            gidx = si_v[pl.ds(pos, 16)][0]
            old = sv_v[pl.ds(pos, 16)]
            sv_v[pl.ds(pos, 16)] = jnp.where(iota == 0, inf, old)
            v_out = jnp.where(iota == k, m, v_out)
            i_out = jnp.where(iota == k, gidx, i_out)
            return v_out, i_out

        v_out, i_out = lax.fori_loop(
            0, K, extract,
            (jnp.full((16,), jnp.inf, jnp.float32),
             jnp.zeros((16,), jnp.int32)))
        vstage[ql, pl.ds(0, 16)] = v_out
        istage[ql, pl.ds(0, 16)] = i_out
        return 0

    lax.fori_loop(0, qpw, per_query, 0)

    pltpu.async_copy(vstage, vals_hbm.at[pl.ds(qbase, qpw)], sem).wait()
    pltpu.async_copy(istage, idx_hbm.at[pl.ds(qbase, qpw)], sem).wait()


def kernel(queries, positions, scales, quaternions):
    Q = queries.shape[0]
    N = positions.shape[0]
    Rm = _quat_rotmat(quaternions)
    s2inv = 1.0 / (scales * scales + 1e-8)
    A = Rm * s2inv[:, None, :]
    Abf = A.astype(jnp.bfloat16)
    Rbf = Rm.astype(jnp.bfloat16)
    Sinv = jnp.einsum('nij,nkj->nik', Rbf, Abf,
                      preferred_element_type=jnp.float32)
    Sflat_bf = Sinv.astype(jnp.bfloat16).reshape(N, 9)
    bt = [Sinv[:, :, j] * positions[:, None, j] for j in range(3)]
    b = (bt[0] + bt[1]) + bt[2]
    ct = [positions[:, j] * b[:, j] for j in range(3)]
    c = (ct[0] + ct[2]) + ct[1]
    Qfeat = (queries[:, :, None] * queries[:, None, :]).reshape(Q, 9)

    nch = N // CH
    HALVES = 2
    Qh = Q // HALVES
    qpw = Qh // NW

    tc = pl.pallas_call(
        _dist_body,
        grid=(Qh // BQ,),
        in_specs=[
            pl.BlockSpec((BQ, 9), lambda i: (i, 0)),
            pl.BlockSpec((BQ, 3), lambda i: (i, 0)),
            pl.BlockSpec((N, 9), lambda i: (0, 0)),
            pl.BlockSpec((N, 3), lambda i: (0, 0)),
            pl.BlockSpec((1, N), lambda i: (0, 0)),
        ],
        out_specs=[
            pl.BlockSpec((BQ, N), lambda i: (i, 0)),
            pl.BlockSpec((BQ, nch), lambda i: (i, 0)),
        ],
        out_shape=[
            jax.ShapeDtypeStruct((Qh, N), jnp.float32),
            jax.ShapeDtypeStruct((Qh, nch), jnp.float32),
        ],
    )

    sc = functools.partial(
        pl.kernel,
        mesh=plsc.VectorSubcoreMesh(core_axis_name="c", subcore_axis_name="s"),
        compiler_params=pltpu.CompilerParams(needs_layout_passes=False),
        out_type=[jax.ShapeDtypeStruct((Qh, 16), jnp.float32),
                  jax.ShapeDtypeStruct((Qh, 16), jnp.int32)],
        scratch_types=[
            pltpu.VMEM((qpw, nch), jnp.float32),
            pltpu.VMEM((nch,), jnp.float32),
            pltpu.VMEM((nch + 16,), jnp.int32),
            pltpu.VMEM((nch + 16,), jnp.int32),
            pltpu.VMEM((16, CH), jnp.float32),
            pltpu.VMEM((SVCAP,), jnp.float32),
            pltpu.VMEM((SVCAP,), jnp.int32),
            pltpu.VMEM((qpw, 16), jnp.float32),
            pltpu.VMEM((qpw, 16), jnp.int32),
            pltpu.SemaphoreType.DMA,
            pltpu.SemaphoreType.DMA,
        ],
    )(_sc_topk_body)

    halves = [tc(Qfeat[h * Qh:(h + 1) * Qh], queries[h * Qh:(h + 1) * Qh],
                 Sflat_bf, b, c[None, :]) for h in range(HALVES)]
    outs = [sc(d.reshape(Qh * nch, CH), m) for d, m in halves]
    vals = jnp.concatenate([o[0][:, :K] for o in outs], axis=0)
    idx = jnp.concatenate([o[1][:, :K] for o in outs], axis=0)
    return vals, idx
